# Initial kernel scaffold; baseline (speedup 1.0000x reference)
import jax
import jax.numpy as jnp
from jax.experimental import pallas as pl


def kernel(x, L_tilde, W1, b1, W2, b2, gamma):
    raise NotImplementedError("write your pallas kernel here")


# speedup vs baseline: 1.2342x; 1.2342x over previous
import functools

import jax
import jax.numpy as jnp
from jax.experimental import pallas as pl
from jax.experimental.pallas import tpu as pltpu

KHOPS = 8


def _pick_blk(n: int) -> int:
    for b in (400, 200, 100, 16, 8):
        if n % b == 0:
            return b
    return n


def _encoder_body(x_ref, w1_ref, b1_ref, w2_ref, b2_ref, g_ref, h_ref, z0_ref):
    h1 = jnp.dot(x_ref[:], w1_ref[:], preferred_element_type=jnp.float32)
    h1 = jnp.maximum(h1 + b1_ref[:], 0.0)
    h = jnp.dot(h1, w2_ref[:], preferred_element_type=jnp.float32) + b2_ref[:]
    h_ref[:] = h
    z0_ref[:] = g_ref[0:1, :] * h


def _hop1_body(l_ref, h_ref, z0_ref, g_ref, lbf_ref, t1_ref, z_ref):
    lb = l_ref[:].astype(jnp.bfloat16)
    lbf_ref[:] = lb
    t1 = jnp.dot(lb, h_ref[:].astype(jnp.bfloat16),
                 preferred_element_type=jnp.float32)
    t1_ref[:] = t1
    z_ref[:] = z0_ref[:] + g_ref[1:2, :] * t1


def _hop_body(l_ref, tc_ref, tp_ref, z_ref, g_ref, tn_ref, zo_ref, *, k):
    acc = jnp.dot(l_ref[:], tc_ref[:].astype(jnp.bfloat16),
                  preferred_element_type=jnp.float32)
    tn = 2.0 * acc - tp_ref[:]
    tn_ref[:] = tn
    zo_ref[:] = z_ref[:] + g_ref[k:k + 1, :] * tn


def kernel(x, L_tilde, W1, b1, W2, b2, gamma):
    n, in_dim = x.shape
    hid = W1.shape[1]
    f = W2.shape[1]
    blk = _pick_blk(n)
    nblk = n // blk

    g = jnp.broadcast_to(gamma[:, None], (KHOPS + 1, f)).astype(jnp.float32)
    b1r = b1.reshape(1, hid)
    b2r = b2.reshape(1, f)

    h, z = pl.pallas_call(
        _encoder_body,
        out_shape=[
            jax.ShapeDtypeStruct((n, f), jnp.float32),
            jax.ShapeDtypeStruct((n, f), jnp.float32),
        ],
    )(x, W1, b1r, W2, b2r, g)

    row_spec_f = pl.BlockSpec((blk, f), lambda i: (i, 0))
    full_t_spec = pl.BlockSpec((n, f), lambda i: (0, 0))
    g_spec = pl.BlockSpec((KHOPS + 1, f), lambda i: (0, 0))

    lbf, t1, z = pl.pallas_call(
        _hop1_body,
        grid=(nblk,),
        in_specs=[
            pl.BlockSpec((blk, n), lambda i: (i, 0)),
            full_t_spec,
            row_spec_f,
            g_spec,
        ],
        out_specs=[
            pl.BlockSpec((blk, n), lambda i: (i, 0)),
            row_spec_f,
            row_spec_f,
        ],
        out_shape=[
            jax.ShapeDtypeStruct((n, n), jnp.bfloat16),
            jax.ShapeDtypeStruct((n, f), jnp.float32),
            jax.ShapeDtypeStruct((n, f), jnp.float32),
        ],
        compiler_params=pltpu.CompilerParams(
            dimension_semantics=("parallel",)),
    )(L_tilde, h, z, g)

    t_prev, t_curr = h, t1
    for k in range(2, KHOPS + 1):
        t_next, z = pl.pallas_call(
            functools.partial(_hop_body, k=k),
            grid=(nblk,),
            in_specs=[
                pl.BlockSpec((blk, n), lambda i: (i, 0)),
                full_t_spec,
                row_spec_f,
                row_spec_f,
                g_spec,
            ],
            out_specs=[row_spec_f, row_spec_f],
            out_shape=[
                jax.ShapeDtypeStruct((n, f), jnp.float32),
                jax.ShapeDtypeStruct((n, f), jnp.float32),
            ],
            compiler_params=pltpu.CompilerParams(
                dimension_semantics=("parallel",)),
        )(lbf, t_curr, t_prev, z, g)
        t_prev, t_curr = t_curr, t_next
    return z


# speedup vs baseline: 1.3427x; 1.0879x over previous
import functools

import jax
import jax.numpy as jnp
from jax.experimental import pallas as pl
from jax.experimental.pallas import tpu as pltpu

KHOPS = 8


def _pick_blk(n: int) -> int:
    for b in (400, 200, 100, 16, 8):
        if n % b == 0:
            return b
    return n


def _encoder_body(x_ref, w1_ref, b1_ref, w2_ref, b2_ref, g_ref, h_ref, z0_ref):
    h1 = jnp.dot(x_ref[:], w1_ref[:], preferred_element_type=jnp.float32)
    h1 = jnp.maximum(h1 + b1_ref[:], 0.0)
    h = jnp.dot(h1, w2_ref[:], preferred_element_type=jnp.float32) + b2_ref[:]
    h_ref[:] = h
    z0_ref[:] = g_ref[0:1, :] * h


def _hop1_body(l_ref, h_ref, z0_ref, g_ref, lbf_ref, t1_ref, z_ref):
    lb = l_ref[:].astype(jnp.bfloat16)
    lbf_ref[:] = lb
    t1 = jnp.dot(lb, h_ref[:].astype(jnp.bfloat16),
                 preferred_element_type=jnp.float32)
    t1_ref[:] = t1
    z_ref[:] = z0_ref[:] + g_ref[1:2, :] * t1


def _hops_body(l_ref, h_ref, t1_ref, z_ref, g_ref, zo_ref,
               tp_s, tc_s, tcb_s, tn_s, *, blk):
    hop = pl.program_id(0)
    i = pl.program_id(1)

    @pl.when(jnp.logical_and(hop == 0, i == 0))
    def _init():
        tp_s[:] = h_ref[:]
        tc_s[:] = t1_ref[:]
        tcb_s[:] = t1_ref[:].astype(jnp.bfloat16)
        zo_ref[:] = z_ref[:]

    @pl.when(jnp.logical_and(hop > 0, i == 0))
    def _rotate():
        tp_s[:] = tc_s[:]
        tc_s[:] = tn_s[:]
        tcb_s[:] = tn_s[:].astype(jnp.bfloat16)

    rows = pl.ds(i * blk, blk)
    acc = jnp.dot(l_ref[:], tcb_s[:], preferred_element_type=jnp.float32)
    tn = 2.0 * acc - tp_s[rows, :]
    tn_s[rows, :] = tn
    gk = g_ref[pl.ds(hop + 2, 1), :]
    zo_ref[rows, :] += gk * tn


def kernel(x, L_tilde, W1, b1, W2, b2, gamma):
    n, in_dim = x.shape
    hid = W1.shape[1]
    f = W2.shape[1]
    blk = _pick_blk(n)
    nblk = n // blk

    g = jnp.broadcast_to(gamma[:, None], (KHOPS + 1, f)).astype(jnp.float32)
    b1r = b1.reshape(1, hid)
    b2r = b2.reshape(1, f)

    h, z = pl.pallas_call(
        _encoder_body,
        out_shape=[
            jax.ShapeDtypeStruct((n, f), jnp.float32),
            jax.ShapeDtypeStruct((n, f), jnp.float32),
        ],
    )(x, W1, b1r, W2, b2r, g)

    row_spec_f = pl.BlockSpec((blk, f), lambda i: (i, 0))
    full_t_spec = pl.BlockSpec((n, f), lambda i: (0, 0))
    g_spec = pl.BlockSpec((KHOPS + 1, f), lambda i: (0, 0))

    lbf, t1, z = pl.pallas_call(
        _hop1_body,
        grid=(nblk,),
        in_specs=[
            pl.BlockSpec((blk, n), lambda i: (i, 0)),
            full_t_spec,
            row_spec_f,
            g_spec,
        ],
        out_specs=[
            pl.BlockSpec((blk, n), lambda i: (i, 0)),
            row_spec_f,
            row_spec_f,
        ],
        out_shape=[
            jax.ShapeDtypeStruct((n, n), jnp.bfloat16),
            jax.ShapeDtypeStruct((n, f), jnp.float32),
            jax.ShapeDtypeStruct((n, f), jnp.float32),
        ],
        compiler_params=pltpu.CompilerParams(
            dimension_semantics=("parallel",)),
    )(L_tilde, h, z, g)

    full_t2 = pl.BlockSpec((n, f), lambda h_, i: (0, 0))
    z = pl.pallas_call(
        functools.partial(_hops_body, blk=blk),
        grid=(KHOPS - 1, nblk),
        in_specs=[
            pl.BlockSpec((blk, n), lambda h_, i: (i, 0)),
            full_t2,
            full_t2,
            full_t2,
            pl.BlockSpec((KHOPS + 1, f), lambda h_, i: (0, 0)),
        ],
        out_specs=full_t2,
        out_shape=jax.ShapeDtypeStruct((n, f), jnp.float32),
        scratch_shapes=[
            pltpu.VMEM((n, f), jnp.float32),
            pltpu.VMEM((n, f), jnp.float32),
            pltpu.VMEM((n, f), jnp.bfloat16),
            pltpu.VMEM((n, f), jnp.float32),
        ],
        compiler_params=pltpu.CompilerParams(
            dimension_semantics=("arbitrary", "arbitrary")),
    )(lbf, h, t1, z, g)
    return z


# speedup vs baseline: 1.3513x; 1.0064x over previous
import functools

import jax
import jax.numpy as jnp
from jax.experimental import pallas as pl
from jax.experimental.pallas import tpu as pltpu

KHOPS = 8


def _pick_blk(n: int) -> int:
    for b in (400, 200, 100, 16, 8):
        if n % b == 0:
            return b
    return n


def _encoder_body(x_ref, w1_ref, b1_ref, w2_ref, b2_ref, g_ref, h_ref, z0_ref):
    h1 = jnp.dot(x_ref[:], w1_ref[:], preferred_element_type=jnp.float32)
    h1 = jnp.maximum(h1 + b1_ref[:], 0.0)
    h = jnp.dot(h1, w2_ref[:], preferred_element_type=jnp.float32) + b2_ref[:]
    h_ref[:] = h
    z0_ref[:] = g_ref[0:1, :] * h


def _hop1_body(l_ref, h_ref, z0_ref, g_ref, lbf_ref, t1_ref, z_ref):
    lb = l_ref[:].astype(jnp.bfloat16)
    lbf_ref[:] = lb
    t1 = jnp.dot(lb, h_ref[:].astype(jnp.bfloat16),
                 preferred_element_type=jnp.float32)
    t1_ref[:] = t1
    z_ref[:] = z0_ref[:] + g_ref[1:2, :] * t1


def _hops_body(l_ref, h_ref, t1_ref, z_ref, g_ref, zo_ref, tbf_s, *, blk):
    hop = pl.program_id(0)
    i = pl.program_id(1)

    @pl.when(jnp.logical_and(hop == 0, i == 0))
    def _init():
        tbf_s[0] = h_ref[:].astype(jnp.bfloat16)
        tbf_s[1] = t1_ref[:].astype(jnp.bfloat16)
        zo_ref[:] = z_ref[:]

    ip = hop % 3
    ic = (hop + 1) % 3
    it = (hop + 2) % 3
    rows = pl.ds(i * blk, blk)
    acc = jnp.dot(l_ref[:], tbf_s[ic], preferred_element_type=jnp.float32)
    tn = 2.0 * acc - tbf_s[ip, rows, :].astype(jnp.float32)
    tbf_s[it, rows, :] = tn.astype(jnp.bfloat16)
    gk = g_ref[pl.ds(hop + 2, 1), :]
    zo_ref[rows, :] += gk * tn


def kernel(x, L_tilde, W1, b1, W2, b2, gamma):
    n, in_dim = x.shape
    hid = W1.shape[1]
    f = W2.shape[1]
    blk = _pick_blk(n)
    nblk = n // blk

    g = jnp.broadcast_to(gamma[:, None], (KHOPS + 1, f)).astype(jnp.float32)
    b1r = b1.reshape(1, hid)
    b2r = b2.reshape(1, f)

    h, z = pl.pallas_call(
        _encoder_body,
        out_shape=[
            jax.ShapeDtypeStruct((n, f), jnp.float32),
            jax.ShapeDtypeStruct((n, f), jnp.float32),
        ],
    )(x, W1, b1r, W2, b2r, g)

    row_spec_f = pl.BlockSpec((blk, f), lambda i: (i, 0))
    full_t_spec = pl.BlockSpec((n, f), lambda i: (0, 0))
    g_spec = pl.BlockSpec((KHOPS + 1, f), lambda i: (0, 0))

    lbf, t1, z = pl.pallas_call(
        _hop1_body,
        grid=(nblk,),
        in_specs=[
            pl.BlockSpec((blk, n), lambda i: (i, 0)),
            full_t_spec,
            row_spec_f,
            g_spec,
        ],
        out_specs=[
            pl.BlockSpec((blk, n), lambda i: (i, 0)),
            row_spec_f,
            row_spec_f,
        ],
        out_shape=[
            jax.ShapeDtypeStruct((n, n), jnp.bfloat16),
            jax.ShapeDtypeStruct((n, f), jnp.float32),
            jax.ShapeDtypeStruct((n, f), jnp.float32),
        ],
        compiler_params=pltpu.CompilerParams(
            dimension_semantics=("parallel",)),
    )(L_tilde, h, z, g)

    full_t2 = pl.BlockSpec((n, f), lambda h_, i: (0, 0))
    z = pl.pallas_call(
        functools.partial(_hops_body, blk=blk),
        grid=(KHOPS - 1, nblk),
        in_specs=[
            pl.BlockSpec((blk, n), lambda h_, i: (i, 0)),
            full_t2,
            full_t2,
            full_t2,
            pl.BlockSpec((KHOPS + 1, f), lambda h_, i: (0, 0)),
        ],
        out_specs=full_t2,
        out_shape=jax.ShapeDtypeStruct((n, f), jnp.float32),
        scratch_shapes=[
            pltpu.VMEM((3, n, f), jnp.bfloat16),
        ],
        compiler_params=pltpu.CompilerParams(
            dimension_semantics=("arbitrary", "arbitrary")),
    )(lbf, h, t1, z, g)
    return z
